# baseline (speedup 1.0000x reference)
import functools

import jax
import jax.numpy as jnp
from jax import lax
from jax.experimental import pallas as pl
from jax.experimental.pallas import tpu as pltpu
from jax.experimental.pallas import tpu_sc as plsc

E = 320000
D = 128
V_SEG = 10000
NC = 2
NS = 16
NW = NC * NS
RW = E // NW
BL = 80
NBL = RW // BL
NBUF = 4
NPER = -(-NBL // NBUF)
VCHUNK = 1000
NVT = V_SEG // VCHUNK


def _sc_partial_segment_sum(H, ids4, zrows):
    mesh = plsc.VectorSubcoreMesh(
        core_axis_name="c", subcore_axis_name="s",
        num_cores=NC, num_subcores=NS)

    @functools.partial(
        pl.kernel,
        out_type=jax.ShapeDtypeStruct((NC, V_SEG, D), jnp.float32),
        mesh=mesh,
        scratch_types=[
            pltpu.VMEM((NBUF, BL, D), jnp.float32),
            pltpu.VMEM((2, NBUF, 1, BL), jnp.int32),
            pltpu.VMEM_SHARED((V_SEG, D), jnp.float32),
            [pltpu.SemaphoreType.DMA] * NBUF,
            [pltpu.SemaphoreType.DMA] * NBUF,
            [pltpu.SemaphoreType.DMA] * 2,
        ],
    )
    def k(h_hbm, ids_hbm, z_hbm, out_hbm, rows_v, ids_v, acc,
          lsems, ssems, isems):
        c = lax.axis_index("c")
        s = lax.axis_index("s")
        wid = c * NS + s
        row_base = wid * RW
        vbase = s * VCHUNK

        @pl.when(s < NVT)
        def _zero():
            pltpu.sync_copy(z_hbm.at[pl.ds(vbase, VCHUNK), :],
                            acc.at[pl.ds(vbase, VCHUNK), :])

        plsc.subcore_barrier()

        def rows_desc(blk, b):
            return pltpu.make_async_copy(
                h_hbm.at[pl.ds(row_base + blk * BL, BL), :],
                rows_v.at[b], lsems[b])

        def ids_desc(p, st):
            return pltpu.make_async_copy(
                ids_hbm.at[wid, pl.ds(p * NBUF, NBUF), :, :],
                ids_v.at[st], isems[st])

        def scatter_desc(b, st):
            return pltpu.make_async_copy(
                rows_v.at[b], acc.at[ids_v.at[st, b, 0]], ssems[b])

        ids_desc(0, 0).start()
        ids_desc(1, 1).start()
        for b in range(NBUF - 1):
            rows_desc(b, b).start()

        def block_step(blk, b, st, first_of_period, traced):
            if first_of_period:
                ids_desc(blk // NBUF, st).wait()
            rows_desc(blk, b).wait()
            scatter_desc(b, st).start(add=True)
            nb = (b + NBUF - 1) % NBUF
            if traced:
                @pl.when(blk >= 1)
                def _drain():
                    scatter_desc(nb, 0).wait()

                @pl.when(blk + NBUF - 1 < NBL)
                def _refill():
                    rows_desc(blk + NBUF - 1, nb).start()
            else:
                if blk >= 1:
                    scatter_desc(nb, 0).wait()
                if blk + NBUF - 1 < NBL:
                    rows_desc(blk + NBUF - 1, nb).start()

        NMAIN = (NPER - 2) // 2 * 2 * NBUF

        @pl.loop(0, NMAIN, step=2 * NBUF)
        def _ring(j):
            for b2 in range(2 * NBUF):
                st = b2 // NBUF
                b = b2 % NBUF
                blk = j + b2
                block_step(blk, b, st, first_of_period=(b == 0), traced=True)
                if b == 1:
                    q = j // NBUF + st

                    @pl.when(q >= 1)
                    def _ids_refill():
                        ids_desc(q + 1, 1 - st).start()

        for blk in range(NMAIN, NBL):
            st = (blk // NBUF) % 2
            b = blk % NBUF
            block_step(blk, b, st, first_of_period=(b == 0), traced=False)
            if blk == NMAIN + 1:
                ids_desc(NPER - 1, (NPER - 1) % 2).start()
        scatter_desc((NBL - 1) % NBUF, 0).wait()
        plsc.subcore_barrier()

        @pl.when(s < NVT)
        def _write():
            pltpu.sync_copy(acc.at[pl.ds(vbase, VCHUNK), :],
                            out_hbm.at[c, pl.ds(vbase, VCHUNK), :])

    return k(H, ids4, zrows)


def _merge_partials(parts):
    BS = 1000

    def body(p_ref, o_ref):
        o_ref[...] = p_ref[0] + p_ref[1]

    return pl.pallas_call(
        body,
        grid=(V_SEG // BS,),
        in_specs=[pl.BlockSpec((NC, BS, D), lambda i: (0, i, 0))],
        out_specs=pl.BlockSpec((BS, D), lambda i: (i, 0)),
        out_shape=jax.ShapeDtypeStruct((V_SEG, D), jnp.float32),
    )(parts)


def kernel(H, X_neis, V):
    del V
    ids3 = X_neis.astype(jnp.int32).reshape(NW, NBL, 1, BL)
    pad = NPER * NBUF - NBL
    ids4 = jnp.concatenate(
        [ids3, jnp.zeros((NW, pad, 1, BL), jnp.int32)], axis=1)
    zrows = jnp.zeros((V_SEG, D), jnp.float32)
    parts = _sc_partial_segment_sum(H, ids4, zrows)
    return _merge_partials(parts)


# speedup vs baseline: 1.0484x; 1.0484x over previous
import functools

import jax
import jax.numpy as jnp
from jax import lax
from jax.experimental import pallas as pl
from jax.experimental.pallas import tpu as pltpu
from jax.experimental.pallas import tpu_sc as plsc

E = 320000
D = 128
V_SEG = 10000
NC = 2
NS = 16
NW = NC * NS
RW = E // NW
BL = 80
NBL = RW // BL
NBUF = 4
VCHUNK = 1000
NVT = V_SEG // VCHUNK


def _sc_partial_segment_sum(H, ids3, zrows):
    mesh = plsc.VectorSubcoreMesh(
        core_axis_name="c", subcore_axis_name="s",
        num_cores=NC, num_subcores=NS)

    @functools.partial(
        pl.kernel,
        out_type=jax.ShapeDtypeStruct((NC, V_SEG, D), jnp.float32),
        mesh=mesh,
        scratch_types=[
            pltpu.VMEM((NBUF, BL, D), jnp.float32),
            pltpu.VMEM((NBUF, 1, BL), jnp.int32),
            pltpu.VMEM_SHARED((V_SEG, D), jnp.float32),
            [pltpu.SemaphoreType.DMA] * NBUF,
            [pltpu.SemaphoreType.DMA] * NBUF,
        ],
    )
    def k(h_hbm, ids_hbm, z_hbm, out_hbm, rows_v, ids_v, acc, lsems, ssems):
        c = lax.axis_index("c")
        s = lax.axis_index("s")
        wid = c * NS + s
        row_base = wid * RW
        vbase = s * VCHUNK

        @pl.when(s < NVT)
        def _zero():
            pltpu.sync_copy(z_hbm.at[pl.ds(vbase, VCHUNK), :],
                            acc.at[pl.ds(vbase, VCHUNK), :])

        plsc.subcore_barrier()

        def load_descs(blk, b):
            rows = pltpu.make_async_copy(
                h_hbm.at[pl.ds(row_base + blk * BL, BL), :],
                rows_v.at[b], lsems[b])
            ids = pltpu.make_async_copy(
                ids_hbm.at[wid, pl.ds(blk, 1), :], ids_v.at[b], lsems[b])
            return rows, ids

        def start_load(blk, b):
            for d in load_descs(blk, b):
                d.start()

        def wait_load(blk, b):
            for d in load_descs(blk, b):
                d.wait()

        def scatter_desc(b):
            return pltpu.make_async_copy(
                rows_v.at[b], acc.at[ids_v.at[b, 0]], ssems[b])

        for b in range(NBUF - 1):
            start_load(b, b)

        @pl.loop(0, NBL - 1, step=NBUF)
        def _ring(j):
            for b in range(NBUF):
                blk = j + b
                wait_load(blk, b)
                scatter_desc(b).start(add=True)
                nb = (b + NBUF - 1) % NBUF

                @pl.when(blk >= 1)
                def _drain():
                    scatter_desc(nb).wait()

                @pl.when(blk + NBUF - 1 < NBL)
                def _refill():
                    start_load(blk + NBUF - 1, nb)

        last = NBL - 1
        lb = last % NBUF
        wait_load(last, lb)
        scatter_desc(lb).start(add=True)
        scatter_desc((lb + NBUF - 1) % NBUF).wait()
        scatter_desc(lb).wait()
        plsc.subcore_barrier()

        @pl.when(s < NVT)
        def _write():
            pltpu.sync_copy(acc.at[pl.ds(vbase, VCHUNK), :],
                            out_hbm.at[c, pl.ds(vbase, VCHUNK), :])

    return k(H, ids3, zrows)


def _merge_partials(parts):
    BS = 1000

    def body(p_ref, o_ref):
        o_ref[...] = p_ref[0] + p_ref[1]

    return pl.pallas_call(
        body,
        grid=(V_SEG // BS,),
        in_specs=[pl.BlockSpec((NC, BS, D), lambda i: (0, i, 0))],
        out_specs=pl.BlockSpec((BS, D), lambda i: (i, 0)),
        out_shape=jax.ShapeDtypeStruct((V_SEG, D), jnp.float32),
    )(parts)


def kernel(H, X_neis, V):
    del V
    ids3 = X_neis.astype(jnp.int32).reshape(NW, NBL, BL)
    zrows = jnp.zeros((V_SEG, D), jnp.float32)
    parts = _sc_partial_segment_sum(H, ids3, zrows)
    return _merge_partials(parts)


# speedup vs baseline: 1.0638x; 1.0147x over previous
import functools

import jax
import jax.numpy as jnp
from jax import lax
from jax.experimental import pallas as pl
from jax.experimental.pallas import tpu as pltpu
from jax.experimental.pallas import tpu_sc as plsc

E = 320000
D = 128
V_SEG = 10000
NC = 2
NS = 16
NW = NC * NS
RW = E // NW
BL = 80
NBL = RW // BL
NBUF = 4
VCHUNK = 1000
NVT = V_SEG // VCHUNK


def _sc_partial_segment_sum(H, ids3, zrows):
    mesh = plsc.VectorSubcoreMesh(
        core_axis_name="c", subcore_axis_name="s",
        num_cores=NC, num_subcores=NS)

    @functools.partial(
        pl.kernel,
        out_type=jax.ShapeDtypeStruct((NC, V_SEG, D), jnp.float32),
        mesh=mesh,
        scratch_types=[
            pltpu.VMEM((NBUF, BL, D), jnp.float32),
            pltpu.VMEM((NBUF, 1, BL), jnp.int32),
            pltpu.VMEM_SHARED((V_SEG, D), jnp.float32),
            [pltpu.SemaphoreType.DMA] * NBUF,
            [pltpu.SemaphoreType.DMA] * NBUF,
        ],
    )
    def k(h_hbm, ids_hbm, z_hbm, out_hbm, rows_v, ids_v, acc, lsems, ssems):
        c = lax.axis_index("c")
        s = lax.axis_index("s")
        wid = c * NS + s
        row_base = wid * RW
        vbase = s * VCHUNK

        def load_descs(blk, b):
            rows = pltpu.make_async_copy(
                h_hbm.at[pl.ds(row_base + blk * BL, BL), :],
                rows_v.at[b], lsems[b])
            ids = pltpu.make_async_copy(
                ids_hbm.at[wid, pl.ds(blk, 1), :], ids_v.at[b], lsems[b])
            return rows, ids

        def start_load(blk, b):
            for d in load_descs(blk, b):
                d.start()

        def wait_load(blk, b):
            for d in load_descs(blk, b):
                d.wait()

        def scatter_desc(b):
            return pltpu.make_async_copy(
                rows_v.at[b], acc.at[ids_v.at[b, 0]], ssems[b])

        for b in range(NBUF - 1):
            start_load(b, b)

        @pl.when(s < NVT)
        def _zero():
            pltpu.sync_copy(z_hbm.at[pl.ds(vbase, VCHUNK), :],
                            acc.at[pl.ds(vbase, VCHUNK), :])

        plsc.subcore_barrier()

        @pl.loop(0, NBL - 1, step=NBUF)
        def _ring(j):
            for b in range(NBUF):
                blk = j + b
                wait_load(blk, b)
                scatter_desc(b).start(add=True)
                nb = (b + NBUF - 1) % NBUF

                @pl.when(blk >= 1)
                def _drain():
                    scatter_desc(nb).wait()

                @pl.when(blk + NBUF - 1 < NBL)
                def _refill():
                    start_load(blk + NBUF - 1, nb)

        last = NBL - 1
        lb = last % NBUF
        wait_load(last, lb)
        scatter_desc(lb).start(add=True)
        scatter_desc((lb + NBUF - 1) % NBUF).wait()
        scatter_desc(lb).wait()
        plsc.subcore_barrier()

        @pl.when(s < NVT)
        def _write():
            pltpu.sync_copy(acc.at[pl.ds(vbase, VCHUNK), :],
                            out_hbm.at[c, pl.ds(vbase, VCHUNK), :])

    return k(H, ids3, zrows)


def _merge_partials(parts):
    BS = 1000

    def body(p_ref, o_ref):
        o_ref[...] = p_ref[0] + p_ref[1]

    return pl.pallas_call(
        body,
        grid=(V_SEG // BS,),
        in_specs=[pl.BlockSpec((NC, BS, D), lambda i: (0, i, 0))],
        out_specs=pl.BlockSpec((BS, D), lambda i: (i, 0)),
        out_shape=jax.ShapeDtypeStruct((V_SEG, D), jnp.float32),
    )(parts)


def kernel(H, X_neis, V):
    del V
    ids3 = X_neis.astype(jnp.int32).reshape(NW, NBL, BL)
    zrows = jnp.zeros((V_SEG, D), jnp.float32)
    parts = _sc_partial_segment_sum(H, ids3, zrows)
    return _merge_partials(parts)
